# baseline (speedup 1.0000x reference)
import functools

import jax
import jax.numpy as jnp
from jax import lax
from jax.experimental import pallas as pl
from jax.experimental.pallas import tpu as pltpu
from jax.experimental.pallas import tpu_sc as plsc

VOCAB = 400002
EMBED = 50
BATCH = 4096
SEQ = 200

NC, NS = 2, 16
NW = NC * NS
NTOT = BATCH * SEQ
NELT = VOCAB * EMBED
NGRAN = (NELT + 12) // 16
CHUNK = 256
NLISTS = CHUNK // 32
PER_W = NTOT // NW
N_CHUNKS = PER_W // CHUNK

_mesh = plsc.VectorSubcoreMesh(
    core_axis_name="c", subcore_axis_name="s", num_cores=NC, num_subcores=NS
)


@functools.partial(
    pl.kernel,
    out_type=jax.ShapeDtypeStruct((NTOT * EMBED,), jnp.float32),
    mesh=_mesh,
    scratch_types=[
        pltpu.VMEM((CHUNK,), jnp.int32),
        pltpu.VMEM((CHUNK,), jnp.int32),
        pltpu.VMEM((CHUNK,), jnp.int32),
        pltpu.VMEM((NLISTS, 128), jnp.int32),
        pltpu.VMEM((4 * CHUNK, 16), jnp.float32),
        pltpu.VMEM((CHUNK * EMBED,), jnp.float32),
        pltpu.SemaphoreType.DMA,
    ],
    compiler_params=pltpu.CompilerParams(
        use_tc_tiling_on_sc=False, needs_layout_passes=False
    ),
)
def _gather_kernel(tab_hbm, idx_hbm, out_hbm, idx_v, g0_v, off_v, gl_v,
                   win_v, dst_v, sem):
    wid = lax.axis_index("s") * NC + lax.axis_index("c")
    ielt0 = wid * PER_W
    oelt0 = wid * PER_W * EMBED
    lane = lax.iota(jnp.int32, 16)
    qv = lane & 3
    rv = lane >> 2

    @pl.loop(0, N_CHUNKS)
    def _chunk(m):
        pltpu.sync_copy(idx_hbm.at[pl.ds(ielt0 + m * CHUNK, CHUNK)], idx_v)

        for t in range(CHUNK // 16):
            iv = idx_v[pl.ds(16 * t, 16)]
            g0_v[pl.ds(16 * t, 16)] = (iv * 25) >> 3
            off_v[pl.ds(16 * t, 16)] = (iv * 2) & 15
        for k in range(NLISTS):
            for w in range(8):
                src = plsc.load_gather(g0_v, [rv + (4 * w + 32 * k)])
                gl_v[k, pl.ds(16 * w, 16)] = src + qv

        copies = [
            pltpu.async_copy(tab_hbm.at[gl_v.at[k]],
                             win_v.at[pl.ds(128 * k, 128)], sem)
            for k in range(NLISTS)
        ]
        for c in copies:
            c.wait()

        for rt in range(CHUNK // 16):
            ovec = off_v[pl.ds(16 * rt, 16)]
            for j in range(16):
                r = 16 * rt + j
                o = ovec[j]
                for q in range(3):
                    dst_v[pl.ds(r * EMBED + 16 * q, 16)] = (
                        win_v[4 * r + q, pl.ds(o, 16)]
                    )
                dst_v[pl.ds(r * EMBED + 34, 16)] = (
                    win_v[4 * r + 2, pl.ds(o + 2, 16)]
                )

        pltpu.sync_copy(
            dst_v,
            out_hbm.at[pl.ds(oelt0 + m * CHUNK * EMBED, CHUNK * EMBED)],
        )


def kernel(input_ids, word_embeddings):
    tab16 = jnp.pad(word_embeddings.reshape(-1), (0, 16 * NGRAN - NELT))
    tab16 = tab16.reshape(NGRAN, 16)
    idx1d = input_ids.reshape(-1).astype(jnp.int32)
    out = _gather_kernel(tab16, idx1d)
    return out.reshape(BATCH, SEQ, EMBED)


# speedup vs baseline: 1.4155x; 1.4155x over previous
import functools

import jax
import jax.numpy as jnp
from jax import lax
from jax.experimental import pallas as pl
from jax.experimental.pallas import tpu as pltpu
from jax.experimental.pallas import tpu_sc as plsc

VOCAB = 400002
EMBED = 50
EMBED_P = 56
BATCH = 4096
SEQ = 200

NC, NS = 2, 16
NW = NC * NS
ROWS_PER_W = BATCH // NW
BPC = 2
RPC = BPC * SEQ
N_CHUNKS = ROWS_PER_W // BPC
SPLIT = (96, 104)

_mesh = plsc.VectorSubcoreMesh(
    core_axis_name="c", subcore_axis_name="s", num_cores=NC, num_subcores=NS
)


@functools.partial(
    pl.kernel,
    out_type=jax.ShapeDtypeStruct((BATCH, SEQ, EMBED), jnp.float32),
    mesh=_mesh,
    scratch_types=[
        pltpu.VMEM((BPC, SEQ), jnp.int32),
        pltpu.VMEM((RPC, EMBED_P), jnp.float32),
        pltpu.VMEM((BPC, SEQ, EMBED), jnp.float32),
        pltpu.SemaphoreType.DMA,
    ],
    compiler_params=pltpu.CompilerParams(use_tc_tiling_on_sc=False),
)
def _gather_kernel(table_hbm, idx_hbm, out_hbm, idx_v, win_v, dst_v, sem):
    wid = lax.axis_index("s") * NC + lax.axis_index("c")
    brow0 = wid * ROWS_PER_W

    @pl.loop(0, N_CHUNKS)
    def _chunk(m):
        base = brow0 + m * BPC
        pltpu.sync_copy(idx_hbm.at[pl.ds(base, BPC)], idx_v)
        copies = []
        for j in range(BPC):
            off = 0
            for w in SPLIT:
                copies.append(
                    pltpu.async_copy(
                        table_hbm.at[idx_v.at[j, pl.ds(off, w)]],
                        win_v.at[pl.ds(j * SEQ + off, w)],
                        sem,
                    )
                )
                off += w
        for c in copies:
            c.wait()

        for j in range(BPC):
            for s in range(SEQ):
                for st in (0, 16, 32, 34):
                    dst_v[j, s, pl.ds(st, 16)] = (
                        win_v[j * SEQ + s, pl.ds(st, 16)]
                    )

        pltpu.sync_copy(dst_v, out_hbm.at[pl.ds(base, BPC)])


def kernel(input_ids, word_embeddings):
    table_p = jnp.pad(word_embeddings, ((0, 0), (0, EMBED_P - EMBED)))
    return _gather_kernel(table_p, input_ids.astype(jnp.int32))


# speedup vs baseline: 1.6354x; 1.1554x over previous
import functools

import jax
import jax.numpy as jnp
from jax import lax
from jax.experimental import pallas as pl
from jax.experimental.pallas import tpu as pltpu
from jax.experimental.pallas import tpu_sc as plsc

VOCAB = 400002
EMBED = 50
EMBED_P = 56
BATCH = 4096
SEQ = 200

NC, NS = 2, 16
NW = NC * NS
NTOT = BATCH * SEQ
IDX_W = 128
GPC = 8
CHUNK = GPC * IDX_W
PER_W = NTOT // NW
N_CHUNKS = PER_W // CHUNK
IDX_ROWS_PER_W = PER_W // IDX_W

_mesh = plsc.VectorSubcoreMesh(
    core_axis_name="c", subcore_axis_name="s", num_cores=NC, num_subcores=NS
)


@functools.partial(
    pl.kernel,
    out_type=jax.ShapeDtypeStruct((NTOT, EMBED_P), jnp.float32),
    mesh=_mesh,
    scratch_types=[
        pltpu.VMEM((GPC, IDX_W), jnp.int32),
        pltpu.VMEM((CHUNK, EMBED_P), jnp.float32),
        pltpu.SemaphoreType.DMA,
    ],
    compiler_params=pltpu.CompilerParams(use_tc_tiling_on_sc=False),
)
def _gather_kernel(table_hbm, idx_hbm, out_hbm, idx_v, rows_v, sem):
    wid = lax.axis_index("s") * NC + lax.axis_index("c")
    irow0 = wid * IDX_ROWS_PER_W
    orow0 = wid * PER_W

    @pl.loop(0, N_CHUNKS)
    def _chunk(m):
        pltpu.sync_copy(idx_hbm.at[pl.ds(irow0 + m * GPC, GPC)], idx_v)
        copies = [
            pltpu.async_copy(
                table_hbm.at[idx_v.at[g]],
                rows_v.at[pl.ds(g * IDX_W, IDX_W)],
                sem,
            )
            for g in range(GPC)
        ]
        for c in copies:
            c.wait()
        pltpu.sync_copy(rows_v, out_hbm.at[pl.ds(orow0 + m * CHUNK, CHUNK)])


def kernel(input_ids, word_embeddings):
    idx2d = input_ids.reshape(NTOT // IDX_W, IDX_W).astype(jnp.int32)
    table_p = jnp.pad(word_embeddings, ((0, 0), (0, EMBED_P - EMBED)))
    out = _gather_kernel(table_p, idx2d)
    return out[:, :EMBED].reshape(BATCH, SEQ, EMBED)
